# Initial kernel scaffold; baseline (speedup 1.0000x reference)
import jax
import jax.numpy as jnp
from jax.experimental import pallas as pl


def kernel(x_transaction, x_account, x_device, x_ip, x_email, e_by, e_uses, e_from_ip, e_with_email, e_rev_by, e_rev_uses, e_rev_from_ip, e_rev_with_email, Wl0_by, bl0_by, Wr0_by, Wl0_uses, bl0_uses, Wr0_uses, Wl0_from_ip, bl0_from_ip, Wr0_from_ip, Wl0_with_email, bl0_with_email, Wr0_with_email, Wl0_rev_by, bl0_rev_by, Wr0_rev_by, Wl0_rev_uses, bl0_rev_uses, Wr0_rev_uses, Wl0_rev_from_ip, bl0_rev_from_ip, Wr0_rev_from_ip, Wl0_rev_with_email, bl0_rev_with_email, Wr0_rev_with_email, Wl1_by, bl1_by, Wr1_by, Wl1_uses, bl1_uses, Wr1_uses, Wl1_from_ip, bl1_from_ip, Wr1_from_ip, Wl1_with_email, bl1_with_email, Wr1_with_email, Wl1_rev_by, bl1_rev_by, Wr1_rev_by, Wl1_rev_uses, bl1_rev_uses, Wr1_rev_uses, Wl1_rev_from_ip, bl1_rev_from_ip, Wr1_rev_from_ip, Wl1_rev_with_email, bl1_rev_with_email, Wr1_rev_with_email):
    raise NotImplementedError("write your pallas kernel here")


# speedup vs baseline: 3.7775x; 3.7775x over previous
import jax
import jax.numpy as jnp
from jax import lax
from jax.experimental import pallas as pl
from jax.experimental.pallas import tpu as pltpu
from jax.experimental.pallas import tpu_sc as plsc

D = 128
E = 50000
NTX = 50000
NSM = 20000
NC, NS, L = 2, 16, 16
EP = 50176
EPT = EP // NS
BIG = 1 << 30

C_T = 6400
C_S = 5120
PAD_T = 8 * C_T
PAD_S = 4 * C_S
ACC_ROWS = C_T + 256
B = 128
NBMAX = 26

RELS_SC = [
    (0, 1, C_S, 4),
    (0, 2, C_S, 4),
    (0, 3, C_S, 4),
    (0, 4, C_S, 4),
    (1, 0, C_T, 8),
    (2, 0, C_T, 8),
    (3, 0, C_T, 8),
    (4, 0, C_T, 8),
]
DST_GROUPS = {0: [4, 5, 6, 7], 1: [0], 2: [1], 3: [2], 4: [3]}
N_ROWS = {0: NTX, 1: NSM, 2: NSM, 3: NSM, 4: NSM}


def _sc_agg(with_counts):
  out_type = [
      jax.ShapeDtypeStruct((PAD_T if di == 0 else PAD_S, D), jnp.float32)
      for (_, di, _, _) in RELS_SC
  ]
  if with_counts:
    out_type += [
        jax.ShapeDtypeStruct((PAD_T if di == 0 else PAD_S, 16), jnp.float32)
        for (_, di, _, _) in RELS_SC
    ]
  mesh = plsc.VectorSubcoreMesh(
      core_axis_name="c", subcore_axis_name="s", num_cores=NC, num_subcores=NS)
  scratch = [
      pltpu.VMEM((EPT,), jnp.int32),
      pltpu.VMEM((EPT,), jnp.int32),
      pltpu.VMEM((NBMAX, B), jnp.int32),
      pltpu.VMEM((NBMAX, B), jnp.int32),
      pltpu.VMEM((B, D), jnp.float32),
      pltpu.VMEM((32, D), jnp.float32),
      pltpu.VMEM((1, 32), jnp.int32),
      pltpu.VMEM((1, B), jnp.int32),
      pltpu.VMEM((32, 16), jnp.float32),
      pltpu.SemaphoreType.DMA,
      pltpu.VMEM_SHARED((ACC_ROWS, D), jnp.float32),
  ]

  def body(*refs):
    xs_refs = refs[0:5]
    esrc_refs = refs[5:13]
    edst_refs = refs[13:21]
    agg_refs = refs[21:29]
    if with_counts:
      cnt_refs = refs[29:37]
      (src_all, dst_all, csrc, cdst, rows, zero128,
       zidx, widx, crows, sem, acc) = refs[37:]
    else:
      (src_all, dst_all, csrc, cdst, rows, zero128,
       zidx, widx, crows, sem, acc) = refs[29:]

    c = lax.axis_index("c")
    s = lax.axis_index("s")
    iota = lax.iota(jnp.int32, L)
    ione = jnp.ones((L,), jnp.int32)
    izero = jnp.zeros((L,), jnp.int32)
    zf = jnp.zeros((L,), jnp.float32)
    of = jnp.ones((L,), jnp.float32)

    def init_z(i, _):
      for q in range(D // L):
        zero128[i, pl.ds(q * L, L)] = zf
      return 0
    lax.fori_loop(0, 32, init_z, 0)

    fill_src = iota * 64 + s * 16

    for r, (si, di, C, nch) in enumerate(RELS_SC):
      xs = xs_refs[si]
      aggr = agg_refs[r]
      cntr = cnt_refs[r] if with_counts else None
      pltpu.sync_copy(esrc_refs[r].at[pl.ds(s * EPT, EPT)], src_all)
      pltpu.sync_copy(edst_refs[r].at[pl.ds(s * EPT, EPT)], dst_all)
      wrows = C // NS
      zrows = (C + 256) // NS
      nz = -(-zrows // 32)
      nwb_full = wrows // B
      wb_tail = wrows - nwb_full * B
      ncw_full = wrows // 32
      cw_tail = wrows - ncw_full * 32
      fill_dst = C + (iota & 7)

      def pass_body(jj, _):
        chunk = c + NC * jj
        lo = chunk * C
        zbase = s * zrows

        def zb(t, _):
          base = zbase + t * 32
          zidx[0, pl.ds(0, L)] = base + iota
          zidx[0, pl.ds(L, L)] = base + L + iota
          pltpu.sync_copy(zero128, acc.at[zidx.at[0]])
          return 0
        lax.fori_loop(0, nz, zb, 0)
        plsc.subcore_barrier()

        def cbody(i, n):
          dv = dst_all[pl.ds(i * L, L)]
          sv = src_all[pl.ds(i * L, L)]
          m = (dv >= lo) & (dv < lo + C)
          mi = jnp.where(m, ione, izero)
          pos = n + plsc.cumsum(mi) - 1
          prow = lax.shift_right_logical(pos, 7)
          pcol = pos & (B - 1)
          plsc.store_scatter(csrc, [prow, pcol], sv, mask=m)
          plsc.store_scatter(cdst, [prow, pcol], dv - lo, mask=m)
          return n + jnp.sum(mi)
        n = lax.fori_loop(0, EPT // L, cbody, jnp.int32(0))

        nb = (n + (B - 1)) // B
        nfill = (nb * B - n + (L - 1)) // L

        def fbody(t, _):
          base = n + t * L
          brow = lax.shift_right_logical(base, 7)
          bcol = base & (B - 1)
          csrc[brow, pl.ds(bcol, L)] = fill_src
          cdst[brow, pl.ds(bcol, L)] = fill_dst
          return 0
        lax.fori_loop(0, nfill, fbody, 0)

        def bbody(b, _):
          pltpu.async_copy(xs.at[csrc.at[b]], rows, sem).wait()
          pltpu.sync_copy(rows, acc.at[cdst.at[b]], add=True)
          return 0
        lax.fori_loop(0, nb, bbody, 0)
        plsc.subcore_barrier()

        wbase = s * wrows

        def wb(b, _):
          for q in range(B // L):
            widx[0, pl.ds(q * L, L)] = wbase + b * B + q * L + iota
          pltpu.sync_copy(acc.at[widx.at[0]], rows)
          pltpu.sync_copy(rows, aggr.at[pl.ds(lo + wbase + b * B, B)])
          return 0
        lax.fori_loop(0, nwb_full, wb, 0)
        if wb_tail:
          base = wbase + nwb_full * B
          for q in range(B // L):
            off = min(q * L, wb_tail - L)
            widx[0, pl.ds(q * L, L)] = base + off + iota
          pltpu.sync_copy(acc.at[widx.at[0]], rows)
          pltpu.sync_copy(rows.at[pl.ds(0, wb_tail)],
                          aggr.at[pl.ds(lo + base, wb_tail)])

        if with_counts:
          lax.fori_loop(0, nz, zb, 0)
          plsc.subcore_barrier()

          def fill_ones(i, _):
            for q in range(D // L):
              rows[i, pl.ds(q * L, L)] = of
            return 0
          lax.fori_loop(0, B, fill_ones, 0)

          def obody(b, _):
            pltpu.sync_copy(rows, acc.at[cdst.at[b]], add=True)
            return 0
          lax.fori_loop(0, nb, obody, 0)
          plsc.subcore_barrier()

          def cwb(b, _):
            base2 = wbase + b * 32
            zidx[0, pl.ds(0, L)] = base2 + iota
            zidx[0, pl.ds(L, L)] = base2 + L + iota
            pltpu.sync_copy(acc.at[zidx.at[0]], rows.at[pl.ds(0, 32)])

            def rpk(i2, _):
              crows[i2, pl.ds(0, L)] = rows[i2, pl.ds(0, L)]
              return 0
            lax.fori_loop(0, 32, rpk, 0)
            pltpu.sync_copy(crows, cntr.at[pl.ds(lo + base2, 32)])
            return 0
          lax.fori_loop(0, ncw_full, cwb, 0)
          if cw_tail:
            base2 = wbase + ncw_full * 32
            zidx[0, pl.ds(0, L)] = base2 + iota
            zidx[0, pl.ds(L, L)] = base2 + max(cw_tail - L, 0) + iota
            pltpu.sync_copy(acc.at[zidx.at[0]], rows.at[pl.ds(0, 32)])

            def rpk2(i2, _):
              crows[i2, pl.ds(0, L)] = rows[i2, pl.ds(0, L)]
              return 0
            lax.fori_loop(0, 32, rpk2, 0)
            pltpu.sync_copy(crows.at[pl.ds(0, cw_tail)],
                            cntr.at[pl.ds(lo + base2, cw_tail)])
        plsc.subcore_barrier()
        return 0

      lax.fori_loop(0, nch // NC, pass_body, 0)

  return pl.kernel(body, out_type=tuple(out_type), mesh=mesh,
                   scratch_types=scratch,
                   compiler_params=pltpu.CompilerParams(
                       needs_layout_passes=False))


def _tc_dense(k, n_rows):
  BLK = 1000
  grid = (n_rows // BLK,)
  in_specs = (
      [pl.BlockSpec((BLK, D), lambda i: (i, 0)) for _ in range(k)]
      + [pl.BlockSpec((BLK, 16), lambda i: (i, 0)) for _ in range(k)]
      + [pl.BlockSpec((BLK, D), lambda i: (i, 0))]
      + [pl.BlockSpec((D, D), lambda i: (0, 0)) for _ in range(k)]
      + [pl.BlockSpec((D, D), lambda i: (0, 0))]
      + [pl.BlockSpec((1, D), lambda i: (0, 0))]
  )

  def body(*refs):
    aggs = refs[0:k]
    cnts = refs[k:2 * k]
    x = refs[2 * k]
    wls = refs[2 * k + 1:3 * k + 1]
    wr = refs[3 * k + 1]
    bs = refs[3 * k + 2]
    out = refs[3 * k + 3]
    accv = jnp.dot(x[...], wr[...], preferred_element_type=jnp.float32)
    for i in range(k):
      inv = 1.0 / jnp.maximum(cnts[i][...][:, 0:1], 1.0)
      accv = accv + jnp.dot(aggs[i][...] * inv, wls[i][...],
                            preferred_element_type=jnp.float32)
    out[...] = jnp.maximum(accv + bs[...], 0.0)

  return pl.pallas_call(
      body, grid=grid, in_specs=in_specs,
      out_specs=pl.BlockSpec((BLK, D), lambda i: (i, 0)),
      out_shape=jax.ShapeDtypeStruct((n_rows, D), jnp.float32))


def _dense_layer(aggs, cnts, xs, Wl, bl, Wr):
  new_xs = []
  for t in range(5):
    rels = DST_GROUPS[t]
    k = len(rels)
    wr_sum = Wr[rels[0]]
    b_sum = bl[rels[0]]
    for r in rels[1:]:
      wr_sum = wr_sum + Wr[r]
      b_sum = b_sum + bl[r]
    args = ([aggs[r] for r in rels] + [cnts[r] for r in rels] + [xs[t]]
            + [Wl[r] for r in rels] + [wr_sum, b_sum.reshape(1, D)])
    new_xs.append(_tc_dense(k, N_ROWS[t])(*args))
  return tuple(new_xs)


def kernel(x_transaction, x_account, x_device, x_ip, x_email, e_by, e_uses, e_from_ip, e_with_email, e_rev_by, e_rev_uses, e_rev_from_ip, e_rev_with_email, Wl0_by, bl0_by, Wr0_by, Wl0_uses, bl0_uses, Wr0_uses, Wl0_from_ip, bl0_from_ip, Wr0_from_ip, Wl0_with_email, bl0_with_email, Wr0_with_email, Wl0_rev_by, bl0_rev_by, Wr0_rev_by, Wl0_rev_uses, bl0_rev_uses, Wr0_rev_uses, Wl0_rev_from_ip, bl0_rev_from_ip, Wr0_rev_from_ip, Wl0_rev_with_email, bl0_rev_with_email, Wr0_rev_with_email, Wl1_by, bl1_by, Wr1_by, Wl1_uses, bl1_uses, Wr1_uses, Wl1_from_ip, bl1_from_ip, Wr1_from_ip, Wl1_with_email, bl1_with_email, Wr1_with_email, Wl1_rev_by, bl1_rev_by, Wr1_rev_by, Wl1_rev_uses, bl1_rev_uses, Wr1_rev_uses, Wl1_rev_from_ip, bl1_rev_from_ip, Wr1_rev_from_ip, Wl1_rev_with_email, bl1_rev_with_email, Wr1_rev_with_email):
  xs0 = (x_transaction, x_account, x_device, x_ip, x_email)
  es = (e_by, e_uses, e_from_ip, e_with_email,
        e_rev_by, e_rev_uses, e_rev_from_ip, e_rev_with_email)
  esrcs = tuple(jnp.pad(e[0], (0, EP - E), constant_values=BIG) for e in es)
  edsts = tuple(jnp.pad(e[1], (0, EP - E), constant_values=BIG) for e in es)

  Wl0 = (Wl0_by, Wl0_uses, Wl0_from_ip, Wl0_with_email,
         Wl0_rev_by, Wl0_rev_uses, Wl0_rev_from_ip, Wl0_rev_with_email)
  bl0 = (bl0_by, bl0_uses, bl0_from_ip, bl0_with_email,
         bl0_rev_by, bl0_rev_uses, bl0_rev_from_ip, bl0_rev_with_email)
  Wr0 = (Wr0_by, Wr0_uses, Wr0_from_ip, Wr0_with_email,
         Wr0_rev_by, Wr0_rev_uses, Wr0_rev_from_ip, Wr0_rev_with_email)
  Wl1 = (Wl1_by, Wl1_uses, Wl1_from_ip, Wl1_with_email,
         Wl1_rev_by, Wl1_rev_uses, Wl1_rev_from_ip, Wl1_rev_with_email)
  bl1 = (bl1_by, bl1_uses, bl1_from_ip, bl1_with_email,
         bl1_rev_by, bl1_rev_uses, bl1_rev_from_ip, bl1_rev_with_email)
  Wr1 = (Wr1_by, Wr1_uses, Wr1_from_ip, Wr1_with_email,
         Wr1_rev_by, Wr1_rev_uses, Wr1_rev_from_ip, Wr1_rev_with_email)

  outs0 = _sc_agg(True)(*xs0, *esrcs, *edsts)
  aggs0, cnts = outs0[0:8], outs0[8:16]
  xs1 = _dense_layer(aggs0, cnts, xs0, Wl0, bl0, Wr0)

  aggs1 = _sc_agg(False)(*xs1, *esrcs, *edsts)
  xs2 = _dense_layer(aggs1, cnts, xs1, Wl1, bl1, Wr1)
  return xs2


# speedup vs baseline: 4.1749x; 1.1052x over previous
import jax
import jax.numpy as jnp
from jax import lax
from jax.experimental import pallas as pl
from jax.experimental.pallas import tpu as pltpu
from jax.experimental.pallas import tpu_sc as plsc

D = 128
E = 50000
NTX = 50000
NSM = 20000
NC, NS, L = 2, 16, 16
EP = 50176
EPT = EP // NS
BIG = 1 << 30

C_T = 6400
C_S = 5120
PAD_T = 8 * C_T
PAD_S = 4 * C_S
ACC_ROWS = C_T + 256
B = 64
BSH = 6
NBMAX = 51

RELS_SC = [
    (0, 1, C_S, 4),
    (0, 2, C_S, 4),
    (0, 3, C_S, 4),
    (0, 4, C_S, 4),
    (1, 0, C_T, 8),
    (2, 0, C_T, 8),
    (3, 0, C_T, 8),
    (4, 0, C_T, 8),
]
DST_GROUPS = {0: [4, 5, 6, 7], 1: [0], 2: [1], 3: [2], 4: [3]}
N_ROWS = {0: NTX, 1: NSM, 2: NSM, 3: NSM, 4: NSM}


def _sc_agg(with_counts):
  out_type = [
      jax.ShapeDtypeStruct((PAD_T if di == 0 else PAD_S, D), jnp.float32)
      for (_, di, _, _) in RELS_SC
  ]
  if with_counts:
    out_type += [
        jax.ShapeDtypeStruct((PAD_T if di == 0 else PAD_S, 16), jnp.float32)
        for (_, di, _, _) in RELS_SC
    ]
  mesh = plsc.VectorSubcoreMesh(
      core_axis_name="c", subcore_axis_name="s", num_cores=NC, num_subcores=NS)
  scratch = [
      pltpu.VMEM((EPT,), jnp.int32),
      pltpu.VMEM((EPT,), jnp.int32),
      pltpu.VMEM((NBMAX, B), jnp.int32),
      pltpu.VMEM((NBMAX, B), jnp.int32),
      pltpu.VMEM((B, D), jnp.float32),
      pltpu.VMEM((B, D), jnp.float32),
      pltpu.SemaphoreType.DMA,
      pltpu.VMEM((32, D), jnp.float32),
      pltpu.VMEM((1, 32), jnp.int32),
      pltpu.VMEM((1, B), jnp.int32),
      pltpu.VMEM((32, 16), jnp.float32),
      pltpu.SemaphoreType.DMA,
      pltpu.VMEM_SHARED((ACC_ROWS, D), jnp.float32),
  ]

  def body(*refs):
    xs_refs = refs[0:5]
    esrc_refs = refs[5:13]
    edst_refs = refs[13:21]
    agg_refs = refs[21:29]
    if with_counts:
      cnt_refs = refs[29:37]
      (src_all, dst_all, csrc, cdst, rows, rows2, sem2, zero128,
       zidx, widx, crows, sem, acc) = refs[37:]
    else:
      (src_all, dst_all, csrc, cdst, rows, rows2, sem2, zero128,
       zidx, widx, crows, sem, acc) = refs[29:]

    c = lax.axis_index("c")
    s = lax.axis_index("s")
    iota = lax.iota(jnp.int32, L)
    ione = jnp.ones((L,), jnp.int32)
    izero = jnp.zeros((L,), jnp.int32)
    zf = jnp.zeros((L,), jnp.float32)
    of = jnp.ones((L,), jnp.float32)

    def init_z(i, _):
      for q in range(D // L):
        zero128[i, pl.ds(q * L, L)] = zf
      return 0
    lax.fori_loop(0, 32, init_z, 0)

    fill_src = iota * 64 + s * 16

    for r, (si, di, C, nch) in enumerate(RELS_SC):
      xs = xs_refs[si]
      aggr = agg_refs[r]
      cntr = cnt_refs[r] if with_counts else None
      pltpu.sync_copy(esrc_refs[r].at[pl.ds(s * EPT, EPT)], src_all)
      pltpu.sync_copy(edst_refs[r].at[pl.ds(s * EPT, EPT)], dst_all)
      wrows = C // NS
      zrows = (C + 256) // NS
      nz = -(-zrows // 32)
      nwb_full = wrows // B
      wb_tail = wrows - nwb_full * B
      ncw_full = wrows // 32
      cw_tail = wrows - ncw_full * 32
      fill_dst = C + (iota & 7)

      def pass_body(jj, _):
        chunk = c + NC * jj
        lo = chunk * C
        zbase = s * zrows

        def zb(t, _):
          base = zbase + t * 32
          zidx[0, pl.ds(0, L)] = base + iota
          zidx[0, pl.ds(L, L)] = base + L + iota
          pltpu.sync_copy(zero128, acc.at[zidx.at[0]])
          return 0
        lax.fori_loop(0, nz, zb, 0)
        plsc.subcore_barrier()

        def cbody(i, n):
          dv = dst_all[pl.ds(i * L, L)]
          sv = src_all[pl.ds(i * L, L)]
          m = (dv >= lo) & (dv < lo + C)
          mi = jnp.where(m, ione, izero)
          pos = n + plsc.cumsum(mi) - 1
          prow = lax.shift_right_logical(pos, BSH)
          pcol = pos & (B - 1)
          plsc.store_scatter(csrc, [prow, pcol], sv, mask=m)
          plsc.store_scatter(cdst, [prow, pcol], dv - lo, mask=m)
          return n + jnp.sum(mi)
        n = lax.fori_loop(0, EPT // L, cbody, jnp.int32(0))

        nb = (n + (B - 1)) // B
        nfill = (nb * B - n + (L - 1)) // L

        def fbody(t, _):
          base = n + t * L
          brow = lax.shift_right_logical(base, BSH)
          bcol = base & (B - 1)
          csrc[brow, pl.ds(bcol, L)] = fill_src
          cdst[brow, pl.ds(bcol, L)] = fill_dst
          return 0
        lax.fori_loop(0, nfill, fbody, 0)

        @pl.when(nb > 0)
        def _():
          pltpu.async_copy(xs.at[csrc.at[0]], rows, sem)

        def pair_body(t, _):
          b0 = 2 * t
          b1 = b0 + 1

          @pl.when(b1 < nb)
          def _():
            pltpu.async_copy(xs.at[csrc.at[b1]], rows2, sem2)
          pltpu.make_async_copy(xs.at[csrc.at[b0]], rows, sem).wait()
          pltpu.sync_copy(rows, acc.at[cdst.at[b0]], add=True)

          @pl.when(b1 + 1 < nb)
          def _():
            pltpu.async_copy(xs.at[csrc.at[b1 + 1]], rows, sem)

          @pl.when(b1 < nb)
          def _():
            pltpu.make_async_copy(xs.at[csrc.at[b1]], rows2, sem2).wait()
            pltpu.sync_copy(rows2, acc.at[cdst.at[b1]], add=True)
          return 0
        lax.fori_loop(0, (nb + 1) // 2, pair_body, 0)
        plsc.subcore_barrier()

        wbase = s * wrows

        def wb(b, _):
          for q in range(B // L):
            widx[0, pl.ds(q * L, L)] = wbase + b * B + q * L + iota
          pltpu.sync_copy(acc.at[widx.at[0]], rows)
          pltpu.sync_copy(rows, aggr.at[pl.ds(lo + wbase + b * B, B)])
          return 0
        lax.fori_loop(0, nwb_full, wb, 0)
        if wb_tail:
          base = wbase + nwb_full * B
          for q in range(B // L):
            off = min(q * L, wb_tail - L)
            widx[0, pl.ds(q * L, L)] = base + off + iota
          pltpu.sync_copy(acc.at[widx.at[0]], rows)
          pltpu.sync_copy(rows.at[pl.ds(0, wb_tail)],
                          aggr.at[pl.ds(lo + base, wb_tail)])

        if with_counts:
          lax.fori_loop(0, nz, zb, 0)
          plsc.subcore_barrier()

          def fill_ones(i, _):
            for q in range(D // L):
              rows[i, pl.ds(q * L, L)] = of
            return 0
          lax.fori_loop(0, B, fill_ones, 0)

          def obody(b, _):
            pltpu.sync_copy(rows, acc.at[cdst.at[b]], add=True)
            return 0
          lax.fori_loop(0, nb, obody, 0)
          plsc.subcore_barrier()

          def cwb(b, _):
            base2 = wbase + b * 32
            zidx[0, pl.ds(0, L)] = base2 + iota
            zidx[0, pl.ds(L, L)] = base2 + L + iota
            pltpu.sync_copy(acc.at[zidx.at[0]], rows.at[pl.ds(0, 32)])

            def rpk(i2, _):
              crows[i2, pl.ds(0, L)] = rows[i2, pl.ds(0, L)]
              return 0
            lax.fori_loop(0, 32, rpk, 0)
            pltpu.sync_copy(crows, cntr.at[pl.ds(lo + base2, 32)])
            return 0
          lax.fori_loop(0, ncw_full, cwb, 0)
          if cw_tail:
            base2 = wbase + ncw_full * 32
            zidx[0, pl.ds(0, L)] = base2 + iota
            zidx[0, pl.ds(L, L)] = base2 + max(cw_tail - L, 0) + iota
            pltpu.sync_copy(acc.at[zidx.at[0]], rows.at[pl.ds(0, 32)])

            def rpk2(i2, _):
              crows[i2, pl.ds(0, L)] = rows[i2, pl.ds(0, L)]
              return 0
            lax.fori_loop(0, 32, rpk2, 0)
            pltpu.sync_copy(crows.at[pl.ds(0, cw_tail)],
                            cntr.at[pl.ds(lo + base2, cw_tail)])
        plsc.subcore_barrier()
        return 0

      lax.fori_loop(0, nch // NC, pass_body, 0)

  return pl.kernel(body, out_type=tuple(out_type), mesh=mesh,
                   scratch_types=scratch,
                   compiler_params=pltpu.CompilerParams(
                       needs_layout_passes=False))


def _tc_dense(k, n_rows):
  BLK = 1000
  grid = (n_rows // BLK,)
  in_specs = (
      [pl.BlockSpec((BLK, D), lambda i: (i, 0)) for _ in range(k)]
      + [pl.BlockSpec((BLK, 16), lambda i: (i, 0)) for _ in range(k)]
      + [pl.BlockSpec((BLK, D), lambda i: (i, 0))]
      + [pl.BlockSpec((D, D), lambda i: (0, 0)) for _ in range(k)]
      + [pl.BlockSpec((D, D), lambda i: (0, 0))]
      + [pl.BlockSpec((1, D), lambda i: (0, 0))]
  )

  def body(*refs):
    aggs = refs[0:k]
    cnts = refs[k:2 * k]
    x = refs[2 * k]
    wls = refs[2 * k + 1:3 * k + 1]
    wr = refs[3 * k + 1]
    bs = refs[3 * k + 2]
    out = refs[3 * k + 3]
    accv = jnp.dot(x[...], wr[...], preferred_element_type=jnp.float32)
    for i in range(k):
      inv = 1.0 / jnp.maximum(cnts[i][...][:, 0:1], 1.0)
      accv = accv + jnp.dot(aggs[i][...] * inv, wls[i][...],
                            preferred_element_type=jnp.float32)
    out[...] = jnp.maximum(accv + bs[...], 0.0)

  return pl.pallas_call(
      body, grid=grid, in_specs=in_specs,
      out_specs=pl.BlockSpec((BLK, D), lambda i: (i, 0)),
      out_shape=jax.ShapeDtypeStruct((n_rows, D), jnp.float32))


def _dense_layer(aggs, cnts, xs, Wl, bl, Wr):
  new_xs = []
  for t in range(5):
    rels = DST_GROUPS[t]
    k = len(rels)
    wr_sum = Wr[rels[0]]
    b_sum = bl[rels[0]]
    for r in rels[1:]:
      wr_sum = wr_sum + Wr[r]
      b_sum = b_sum + bl[r]
    args = ([aggs[r] for r in rels] + [cnts[r] for r in rels] + [xs[t]]
            + [Wl[r] for r in rels] + [wr_sum, b_sum.reshape(1, D)])
    new_xs.append(_tc_dense(k, N_ROWS[t])(*args))
  return tuple(new_xs)


def kernel(x_transaction, x_account, x_device, x_ip, x_email, e_by, e_uses, e_from_ip, e_with_email, e_rev_by, e_rev_uses, e_rev_from_ip, e_rev_with_email, Wl0_by, bl0_by, Wr0_by, Wl0_uses, bl0_uses, Wr0_uses, Wl0_from_ip, bl0_from_ip, Wr0_from_ip, Wl0_with_email, bl0_with_email, Wr0_with_email, Wl0_rev_by, bl0_rev_by, Wr0_rev_by, Wl0_rev_uses, bl0_rev_uses, Wr0_rev_uses, Wl0_rev_from_ip, bl0_rev_from_ip, Wr0_rev_from_ip, Wl0_rev_with_email, bl0_rev_with_email, Wr0_rev_with_email, Wl1_by, bl1_by, Wr1_by, Wl1_uses, bl1_uses, Wr1_uses, Wl1_from_ip, bl1_from_ip, Wr1_from_ip, Wl1_with_email, bl1_with_email, Wr1_with_email, Wl1_rev_by, bl1_rev_by, Wr1_rev_by, Wl1_rev_uses, bl1_rev_uses, Wr1_rev_uses, Wl1_rev_from_ip, bl1_rev_from_ip, Wr1_rev_from_ip, Wl1_rev_with_email, bl1_rev_with_email, Wr1_rev_with_email):
  xs0 = (x_transaction, x_account, x_device, x_ip, x_email)
  es = (e_by, e_uses, e_from_ip, e_with_email,
        e_rev_by, e_rev_uses, e_rev_from_ip, e_rev_with_email)
  esrcs = tuple(jnp.pad(e[0], (0, EP - E), constant_values=BIG) for e in es)
  edsts = tuple(jnp.pad(e[1], (0, EP - E), constant_values=BIG) for e in es)

  Wl0 = (Wl0_by, Wl0_uses, Wl0_from_ip, Wl0_with_email,
         Wl0_rev_by, Wl0_rev_uses, Wl0_rev_from_ip, Wl0_rev_with_email)
  bl0 = (bl0_by, bl0_uses, bl0_from_ip, bl0_with_email,
         bl0_rev_by, bl0_rev_uses, bl0_rev_from_ip, bl0_rev_with_email)
  Wr0 = (Wr0_by, Wr0_uses, Wr0_from_ip, Wr0_with_email,
         Wr0_rev_by, Wr0_rev_uses, Wr0_rev_from_ip, Wr0_rev_with_email)
  Wl1 = (Wl1_by, Wl1_uses, Wl1_from_ip, Wl1_with_email,
         Wl1_rev_by, Wl1_rev_uses, Wl1_rev_from_ip, Wl1_rev_with_email)
  bl1 = (bl1_by, bl1_uses, bl1_from_ip, bl1_with_email,
         bl1_rev_by, bl1_rev_uses, bl1_rev_from_ip, bl1_rev_with_email)
  Wr1 = (Wr1_by, Wr1_uses, Wr1_from_ip, Wr1_with_email,
         Wr1_rev_by, Wr1_rev_uses, Wr1_rev_from_ip, Wr1_rev_with_email)

  outs0 = _sc_agg(True)(*xs0, *esrcs, *edsts)
  aggs0, cnts = outs0[0:8], outs0[8:16]
  xs1 = _dense_layer(aggs0, cnts, xs0, Wl0, bl0, Wr0)

  aggs1 = _sc_agg(False)(*xs1, *esrcs, *edsts)
  xs2 = _dense_layer(aggs1, cnts, xs1, Wl1, bl1, Wr1)
  return xs2
